# baseline (speedup 1.0000x reference)
import functools

import jax
import jax.numpy as jnp
from jax import lax
from jax.experimental import pallas as pl
from jax.experimental.pallas import tpu as pltpu
from jax.experimental.pallas import tpu_sc as plsc

N_SRC = 5000
N_TGT = 5000
N_EDGE = 320000
D = 128
TW = 144
ONES_COL = 128

NC = 2
NS = 16
NW = NC * NS
CHUNK = 128
NCPT = 80
NCHUNK_PAD = NW * NCPT
E_PAD = NCHUNK_PAD * CHUNK
N_DUMP = 136
N_ACC = N_TGT + N_DUMP
RPS = N_ACC // NS

MM_BLK = 1000
CB_BLK = 1000


def _mm_body(x_ref, w_ref, o_ref):
    acc = jnp.dot(x_ref[...], w_ref[...], preferred_element_type=jnp.float32,
                  precision=lax.Precision.HIGHEST)
    col = lax.broadcasted_iota(jnp.int32, acc.shape, 1)
    o_ref[...] = acc + (col == ONES_COL).astype(jnp.float32)


_matmul = pl.pallas_call(
    _mm_body,
    grid=(N_SRC // MM_BLK,),
    in_specs=[
        pl.BlockSpec((MM_BLK, D), lambda i: (i, 0)),
        pl.BlockSpec((D, TW), lambda i: (0, 0)),
    ],
    out_specs=pl.BlockSpec((MM_BLK, TW), lambda i: (i, 0)),
    out_shape=jax.ShapeDtypeStruct((N_SRC, TW), jnp.float32),
)


def _edge_aggregate_body(table, src2, dst2, zeros, out, sidx, didx, rows,
                         acc_sh, gsems):
    c = lax.axis_index("c")
    s = lax.axis_index("s")
    wid = s * NC + c
    start = wid * NCPT

    pltpu.sync_copy(zeros, acc_sh.at[pl.ds(s * RPS, RPS)])
    pltpu.sync_copy(src2.at[pl.ds(start, NCPT)], sidx)
    pltpu.sync_copy(dst2.at[pl.ds(start, NCPT)], didx)

    plsc.subcore_barrier()

    pltpu.async_copy(table.at[sidx.at[0]], rows.at[0], gsems.at[0])

    def body2(i, carry):
        j0 = 2 * i
        pltpu.async_copy(table.at[sidx.at[j0 + 1]], rows.at[1], gsems.at[1])
        pltpu.make_async_copy(table.at[sidx.at[j0]], rows.at[0],
                              gsems.at[0]).wait()
        pltpu.sync_copy(rows.at[0], acc_sh.at[didx.at[j0]], add=True)

        @pl.when(i < NCPT // 2 - 1)
        def _():
            pltpu.async_copy(table.at[sidx.at[j0 + 2]], rows.at[0],
                             gsems.at[0])

        pltpu.make_async_copy(table.at[sidx.at[j0 + 1]], rows.at[1],
                              gsems.at[1]).wait()
        pltpu.sync_copy(rows.at[1], acc_sh.at[didx.at[j0 + 1]], add=True)
        return carry

    lax.fori_loop(0, NCPT // 2, body2, 0)

    plsc.subcore_barrier()

    pltpu.sync_copy(acc_sh.at[pl.ds(s * RPS, RPS)],
                    out.at[c, pl.ds(s * RPS, RPS)])


@functools.cache
def _make_edge_aggregate():
    mesh = plsc.VectorSubcoreMesh(
        core_axis_name="c", subcore_axis_name="s",
        num_cores=NC, num_subcores=NS)
    return pl.kernel(
        _edge_aggregate_body,
        out_type=jax.ShapeDtypeStruct((NC, N_ACC, TW), jnp.float32),
        mesh=mesh,
        scratch_types=[
            pltpu.VMEM((NCPT, CHUNK), jnp.int32),
            pltpu.VMEM((NCPT, CHUNK), jnp.int32),
            pltpu.VMEM((2, CHUNK, TW), jnp.float32),
            pltpu.VMEM_SHARED((N_ACC, TW), jnp.float32),
            pltpu.SemaphoreType.DMA((2,)),
        ],
        compiler_params=pltpu.CompilerParams(use_tc_tiling_on_sc=False),
    )


def _combine_body(agg_ref, xs_ref, xt_ref, w_ref, os_ref, ot_ref):
    a = agg_ref[0] + agg_ref[1]
    feat = a[:, :D]
    deg = a[:, ONES_COL] + 1.0
    dis = 1.0 / jnp.sqrt(deg)
    xw_t = jnp.dot(xt_ref[...], w_ref[...],
                   preferred_element_type=jnp.float32,
                   precision=lax.Precision.HIGHEST)
    ot = dis[:, None] * feat + (dis * dis)[:, None] * xw_t
    ot_ref[...] = jnp.maximum(ot, 0.0)
    os_ref[...] = jnp.maximum(xs_ref[:, :D], 0.0)


_combine = pl.pallas_call(
    _combine_body,
    grid=(N_TGT // CB_BLK,),
    in_specs=[
        pl.BlockSpec((NC, CB_BLK, TW), lambda i: (0, i, 0)),
        pl.BlockSpec((CB_BLK, TW), lambda i: (i, 0)),
        pl.BlockSpec((CB_BLK, D), lambda i: (i, 0)),
        pl.BlockSpec((D, D), lambda i: (0, 0)),
    ],
    out_specs=[
        pl.BlockSpec((CB_BLK, D), lambda i: (i, 0)),
        pl.BlockSpec((CB_BLK, D), lambda i: (i, 0)),
    ],
    out_shape=[
        jax.ShapeDtypeStruct((N_SRC, D), jnp.float32),
        jax.ShapeDtypeStruct((N_TGT, D), jnp.float32),
    ],
)


def kernel(edge_index, x_s, x_t, W):
    w_ext = jnp.pad(W, ((0, 0), (0, TW - D)))
    xw = _matmul(x_s, w_ext)
    pad = E_PAD - N_EDGE
    src_dump = jax.lax.rem(jnp.arange(pad, dtype=jnp.int32), N_SRC)
    src2 = jnp.concatenate([edge_index[0], src_dump]).reshape(
        NCHUNK_PAD, CHUNK)
    dump = N_TGT + jax.lax.rem(jnp.arange(pad, dtype=jnp.int32), N_DUMP)
    dst2 = jnp.concatenate([edge_index[1], dump]).reshape(NCHUNK_PAD, CHUNK)
    zeros = jnp.zeros((RPS, TW), jnp.float32)
    agg = _make_edge_aggregate()(xw, src2, dst2, zeros)
    out_s, out_t = _combine(agg, xw, x_t, W)
    return out_s, out_t


# speedup vs baseline: 1.0582x; 1.0582x over previous
import functools

import jax
import jax.numpy as jnp
from jax import lax
from jax.experimental import pallas as pl
from jax.experimental.pallas import tpu as pltpu
from jax.experimental.pallas import tpu_sc as plsc

N_SRC = 5000
N_TGT = 5000
N_EDGE = 320000
D = 128
TW = 144
ONES_COL = 128

NC = 2
NS = 16
NW = NC * NS
CHUNK = 128
NCHUNK = N_EDGE // CHUNK
NCPT = 80
N_ACC = N_TGT + 8
RPS = N_ACC // NS

MM_BLK = 1000
CB_BLK = 1000


def _mm_body(x_ref, w_ref, o_ref):
    acc = jnp.dot(x_ref[...], w_ref[...], preferred_element_type=jnp.float32,
                  precision=lax.Precision.HIGHEST)
    col = lax.broadcasted_iota(jnp.int32, acc.shape, 1)
    o_ref[...] = acc + (col == ONES_COL).astype(jnp.float32)


_matmul = pl.pallas_call(
    _mm_body,
    grid=(N_SRC // MM_BLK,),
    in_specs=[
        pl.BlockSpec((MM_BLK, D), lambda i: (i, 0)),
        pl.BlockSpec((D, TW), lambda i: (0, 0)),
    ],
    out_specs=pl.BlockSpec((MM_BLK, TW), lambda i: (i, 0)),
    out_shape=jax.ShapeDtypeStruct((N_SRC, TW), jnp.float32),
)


def _edge_aggregate_body(table, ei3, zeros, out, sidx, didx, rows,
                         acc_sh, gsems):
    c = lax.axis_index("c")
    s = lax.axis_index("s")
    wid = s * NC + c
    start = wid * NCPT
    is_tail = wid == NW - 1
    n2 = jnp.where(is_tail, (NCHUNK - (NW - 1) * NCPT) // 2, NCPT // 2)
    load_start = jnp.minimum(start, NCHUNK - NCPT)
    off = start - load_start

    pltpu.sync_copy(zeros, acc_sh.at[pl.ds(s * RPS, RPS)])
    pltpu.sync_copy(ei3.at[0, pl.ds(load_start, NCPT)], sidx)
    pltpu.sync_copy(ei3.at[1, pl.ds(load_start, NCPT)], didx)

    plsc.subcore_barrier()

    pltpu.async_copy(table.at[sidx.at[off]], rows.at[0], gsems.at[0])

    def body2(i, carry):
        j0 = 2 * i + off
        pltpu.async_copy(table.at[sidx.at[j0 + 1]], rows.at[1], gsems.at[1])
        pltpu.make_async_copy(table.at[sidx.at[j0]], rows.at[0],
                              gsems.at[0]).wait()
        pltpu.sync_copy(rows.at[0], acc_sh.at[didx.at[j0]], add=True)

        @pl.when(i < n2 - 1)
        def _():
            pltpu.async_copy(table.at[sidx.at[j0 + 2]], rows.at[0],
                             gsems.at[0])

        pltpu.make_async_copy(table.at[sidx.at[j0 + 1]], rows.at[1],
                              gsems.at[1]).wait()
        pltpu.sync_copy(rows.at[1], acc_sh.at[didx.at[j0 + 1]], add=True)
        return carry

    lax.fori_loop(0, n2, body2, 0)

    plsc.subcore_barrier()

    pltpu.sync_copy(acc_sh.at[pl.ds(s * RPS, RPS)],
                    out.at[c, pl.ds(s * RPS, RPS)])


@functools.cache
def _make_edge_aggregate():
    mesh = plsc.VectorSubcoreMesh(
        core_axis_name="c", subcore_axis_name="s",
        num_cores=NC, num_subcores=NS)
    return pl.kernel(
        _edge_aggregate_body,
        out_type=jax.ShapeDtypeStruct((NC, N_ACC, TW), jnp.float32),
        mesh=mesh,
        scratch_types=[
            pltpu.VMEM((NCPT, CHUNK), jnp.int32),
            pltpu.VMEM((NCPT, CHUNK), jnp.int32),
            pltpu.VMEM((2, CHUNK, TW), jnp.float32),
            pltpu.VMEM_SHARED((N_ACC, TW), jnp.float32),
            pltpu.SemaphoreType.DMA((2,)),
        ],
        compiler_params=pltpu.CompilerParams(use_tc_tiling_on_sc=False),
    )


def _combine_body(agg_ref, xs_ref, xt_ref, w_ref, os_ref, ot_ref):
    a = agg_ref[0] + agg_ref[1]
    feat = a[:, :D]
    deg = a[:, ONES_COL] + 1.0
    dis = 1.0 / jnp.sqrt(deg)
    xw_t = jnp.dot(xt_ref[...], w_ref[...],
                   preferred_element_type=jnp.float32,
                   precision=lax.Precision.HIGHEST)
    ot = dis[:, None] * feat + (dis * dis)[:, None] * xw_t
    ot_ref[...] = jnp.maximum(ot, 0.0)
    os_ref[...] = jnp.maximum(xs_ref[:, :D], 0.0)


_combine = pl.pallas_call(
    _combine_body,
    grid=(N_TGT // CB_BLK,),
    in_specs=[
        pl.BlockSpec((NC, CB_BLK, TW), lambda i: (0, i, 0)),
        pl.BlockSpec((CB_BLK, TW), lambda i: (i, 0)),
        pl.BlockSpec((CB_BLK, D), lambda i: (i, 0)),
        pl.BlockSpec((D, D), lambda i: (0, 0)),
    ],
    out_specs=[
        pl.BlockSpec((CB_BLK, D), lambda i: (i, 0)),
        pl.BlockSpec((CB_BLK, D), lambda i: (i, 0)),
    ],
    out_shape=[
        jax.ShapeDtypeStruct((N_SRC, D), jnp.float32),
        jax.ShapeDtypeStruct((N_TGT, D), jnp.float32),
    ],
)


def kernel(edge_index, x_s, x_t, W):
    w_ext = jnp.pad(W, ((0, 0), (0, TW - D)))
    xw = _matmul(x_s, w_ext)
    ei3 = edge_index.reshape(2, NCHUNK, CHUNK)
    zeros = jnp.zeros((RPS, TW), jnp.float32)
    agg = _make_edge_aggregate()(xw, ei3, zeros)
    out_s, out_t = _combine(agg, xw, x_t, W)
    return out_s, out_t


# speedup vs baseline: 1.0675x; 1.0088x over previous
import functools

import jax
import jax.numpy as jnp
from jax import lax
from jax.experimental import pallas as pl
from jax.experimental.pallas import tpu as pltpu
from jax.experimental.pallas import tpu_sc as plsc

N_SRC = 5000
N_TGT = 5000
N_EDGE = 320000
D = 128
TW = 144
ONES_COL = 128

NC = 2
NS = 16
NW = NC * NS
CHUNK = 128
NCHUNK = N_EDGE // CHUNK
NCPT = 80
N_ACC = N_TGT + 8
RPS = N_ACC // NS

MM_BLK = 1000
CB_BLK = 1000


def _mm_body(x_ref, w_ref, o_ref):
    acc = jnp.dot(x_ref[...], w_ref[...], preferred_element_type=jnp.float32,
                  precision=lax.Precision.HIGHEST)
    col = lax.broadcasted_iota(jnp.int32, acc.shape, 1)
    o_ref[...] = acc + (col == ONES_COL).astype(jnp.float32)


_matmul = pl.pallas_call(
    _mm_body,
    grid=(N_SRC // MM_BLK,),
    in_specs=[
        pl.BlockSpec((MM_BLK, D), lambda i: (i, 0)),
        pl.BlockSpec((D, TW), lambda i: (0, 0)),
    ],
    out_specs=pl.BlockSpec((MM_BLK, TW), lambda i: (i, 0)),
    out_shape=jax.ShapeDtypeStruct((N_SRC, TW), jnp.float32),
)


def _edge_aggregate_body(table, ei3, zeros, out, sidx, didx, rows,
                         acc_sh, gsems, ssems):
    c = lax.axis_index("c")
    s = lax.axis_index("s")
    wid = s * NC + c
    start = wid * NCPT
    is_tail = wid == NW - 1
    n2 = jnp.where(is_tail, (NCHUNK - (NW - 1) * NCPT) // 2, NCPT // 2)
    load_start = jnp.minimum(start, NCHUNK - NCPT)
    off = start - load_start

    pltpu.sync_copy(zeros, acc_sh.at[pl.ds(s * RPS, RPS)])
    pltpu.sync_copy(ei3.at[0, pl.ds(load_start, NCPT)], sidx)
    pltpu.sync_copy(ei3.at[1, pl.ds(load_start, NCPT)], didx)

    n = 2 * n2

    plsc.subcore_barrier()

    pltpu.async_copy(table.at[sidx.at[off]], rows.at[0], gsems.at[0])
    pltpu.async_copy(table.at[sidx.at[off + 1]], rows.at[1], gsems.at[1])

    def slot(jj, b, bp):
        j = jj + off
        pltpu.make_async_copy(table.at[sidx.at[j]], rows.at[b],
                              gsems.at[b]).wait()
        pltpu.async_copy(rows.at[b], acc_sh.at[didx.at[j]],
                         ssems.at[b], add=True)

        @pl.when(jj >= 1)
        def _():
            pltpu.make_async_copy(rows.at[bp], acc_sh.at[didx.at[j - 1]],
                                  ssems.at[bp]).wait()

        @pl.when(jj + 2 < n)
        def _():
            pltpu.async_copy(table.at[sidx.at[j + 2]], rows.at[bp],
                             gsems.at[bp])

    def body2(i, carry):
        j0 = 2 * i
        b0 = lax.rem(j0, 3)
        slot(j0, b0, lax.rem(j0 + 2, 3))
        slot(j0 + 1, lax.rem(j0 + 1, 3), b0)
        return carry

    lax.fori_loop(0, n2, body2, 0)

    pltpu.make_async_copy(rows.at[lax.rem(n - 1, 3)],
                          acc_sh.at[didx.at[n - 1 + off]],
                          ssems.at[lax.rem(n - 1, 3)]).wait()

    plsc.subcore_barrier()

    pltpu.sync_copy(acc_sh.at[pl.ds(s * RPS, RPS)],
                    out.at[c, pl.ds(s * RPS, RPS)])


@functools.cache
def _make_edge_aggregate():
    mesh = plsc.VectorSubcoreMesh(
        core_axis_name="c", subcore_axis_name="s",
        num_cores=NC, num_subcores=NS)
    return pl.kernel(
        _edge_aggregate_body,
        out_type=jax.ShapeDtypeStruct((NC, N_ACC, TW), jnp.float32),
        mesh=mesh,
        scratch_types=[
            pltpu.VMEM((NCPT, CHUNK), jnp.int32),
            pltpu.VMEM((NCPT, CHUNK), jnp.int32),
            pltpu.VMEM((3, CHUNK, TW), jnp.float32),
            pltpu.VMEM_SHARED((N_ACC, TW), jnp.float32),
            pltpu.SemaphoreType.DMA((3,)),
            pltpu.SemaphoreType.DMA((3,)),
        ],
        compiler_params=pltpu.CompilerParams(use_tc_tiling_on_sc=False),
    )


def _combine_body(agg_ref, xs_ref, xt_ref, w_ref, os_ref, ot_ref):
    a = agg_ref[0] + agg_ref[1]
    feat = a[:, :D]
    deg = a[:, ONES_COL] + 1.0
    dis = 1.0 / jnp.sqrt(deg)
    xw_t = jnp.dot(xt_ref[...], w_ref[...],
                   preferred_element_type=jnp.float32,
                   precision=lax.Precision.HIGHEST)
    ot = dis[:, None] * feat + (dis * dis)[:, None] * xw_t
    ot_ref[...] = jnp.maximum(ot, 0.0)
    os_ref[...] = jnp.maximum(xs_ref[:, :D], 0.0)


_combine = pl.pallas_call(
    _combine_body,
    grid=(N_TGT // CB_BLK,),
    in_specs=[
        pl.BlockSpec((NC, CB_BLK, TW), lambda i: (0, i, 0)),
        pl.BlockSpec((CB_BLK, TW), lambda i: (i, 0)),
        pl.BlockSpec((CB_BLK, D), lambda i: (i, 0)),
        pl.BlockSpec((D, D), lambda i: (0, 0)),
    ],
    out_specs=[
        pl.BlockSpec((CB_BLK, D), lambda i: (i, 0)),
        pl.BlockSpec((CB_BLK, D), lambda i: (i, 0)),
    ],
    out_shape=[
        jax.ShapeDtypeStruct((N_SRC, D), jnp.float32),
        jax.ShapeDtypeStruct((N_TGT, D), jnp.float32),
    ],
)


def kernel(edge_index, x_s, x_t, W):
    w_ext = jnp.pad(W, ((0, 0), (0, TW - D)))
    xw = _matmul(x_s, w_ext)
    ei3 = edge_index.reshape(2, NCHUNK, CHUNK)
    zeros = jnp.zeros((RPS, TW), jnp.float32)
    agg = _make_edge_aggregate()(xw, ei3, zeros)
    out_s, out_t = _combine(agg, xw, x_t, W)
    return out_s, out_t
